# baseline (speedup 1.0000x reference)
import functools

import jax
import jax.numpy as jnp
from jax import lax
from jax.experimental import pallas as pl
from jax.experimental.pallas import tpu as pltpu
from jax.experimental.pallas import tpu_sc as plsc

L = 16
NW = 32
CH = 128


def _spline_body(W, ND, bpw, nch, u_hbm, grid_hbm, out_hbm,
                 u_v, iq0, iq1, iq2, off_v, w0_v, w1_v, w2_v, w3_v,
                 pairs_v, out_t, sem):
    nc = 2
    pw = W // 2
    wid = lax.axis_index("s") * nc + lax.axis_index("c")
    base = wid * bpw

    pltpu.sync_copy(u_hbm.at[pl.ds(base, bpw)], u_v)

    def wcomp(g, carry):
        sl = pl.ds(g * L, L)
        uu = jnp.clip(u_v[sl], 0.0, 1.0)
        x = uu * jnp.float32(W - 1)
        idx = jnp.minimum(x.astype(jnp.int32), W - 2)
        t = x - idx.astype(jnp.float32)
        t2 = t * t
        t3 = t2 * t
        sixth = jnp.float32(1.0 / 6.0)
        w0 = sixth * (-t3 + 3.0 * t2 - 3.0 * t + 1.0)
        w1 = sixth * (3.0 * t3 - 6.0 * t2 + 4.0)
        w2 = 0.5 * (-t3 + t2 + t) + sixth
        w3 = sixth * t3
        lo = idx == 0
        hi = idx == W - 2
        zero = jnp.zeros_like(w0)
        v0 = jnp.where(lo, 2.0 * w0 + w1, jnp.where(hi, zero, w0))
        v1 = jnp.where(lo, w2 - w0, jnp.where(hi, w0, w1))
        v2 = jnp.where(lo, w3, jnp.where(hi, w1 - w3, w2))
        v3 = jnp.where(lo, zero, jnp.where(hi, w2 + 2.0 * w3, w3))
        r0 = jnp.clip(idx - 1, 0, W - 4)
        q0 = jnp.minimum(lax.shift_right_arithmetic(r0, 1), pw - 3)
        iq0[sl] = q0
        iq1[sl] = q0 + 1
        iq2[sl] = q0 + 2
        off_v[sl] = r0 - 2 * q0
        w0_v[sl] = v0
        w1_v[sl] = v1
        w2_v[sl] = v2
        w3_v[sl] = v3
        return carry

    lax.fori_loop(0, bpw // L, wcomp, 0, unroll=2)

    lane = lax.iota(jnp.int32, L)
    cvecs = [jj * L + lane for jj in range(ND // L)]

    for c in range(nch):
        cb = c * CH
        cps = [
            pltpu.async_copy(
                grid_hbm.at[iq.at[pl.ds(cb, CH)]], pairs_v.at[p], sem)
            for p, iq in enumerate((iq0, iq1, iq2))
        ]
        for cp in cps:
            cp.wait()

        def scomp(g2, carry):
            wsl = pl.ds(cb + g2 * L, L)
            a0 = w0_v[wsl]
            a1 = w1_v[wsl]
            a2 = w2_v[wsl]
            a3 = w3_v[wsl]
            ov = off_v[wsl]
            for j in range(L):
                b = g2 * L + j
                s0, s1, s2, s3 = a0[j], a1[j], a2[j], a3[j]
                o = ov[j]
                rows = []
                for k in range(4):
                    ok = o + k
                    rows.append((lax.shift_right_arithmetic(ok, 1),
                                 lax.bitwise_and(ok, 1) * ND))
                for jj in range(ND // L):
                    acc = (
                        pairs_v[rows[0][0], b, pl.ds(rows[0][1] + jj * L, L)]
                        * s0
                        + pairs_v[rows[1][0], b, pl.ds(rows[1][1] + jj * L, L)]
                        * s1
                        + pairs_v[rows[2][0], b, pl.ds(rows[2][1] + jj * L, L)]
                        * s2
                        + pairs_v[rows[3][0], b, pl.ds(rows[3][1] + jj * L, L)]
                        * s3)
                    plsc.store_scatter(out_t, [cvecs[jj], lane * 0 + b], acc)
            return carry

        lax.fori_loop(0, CH // L, scomp, 0)
        rb = wid * nch + c
        for cb8 in range(ND // 8):
            pltpu.sync_copy(
                out_t.at[pl.ds(cb8 * 8, 8), pl.ds(0, CH)],
                out_hbm.at[pl.ds((cb8 * (NW * nch) + rb) * 8, 8), :])


def kernel(u, grid):
    B = u.shape[0]
    W, ND = grid.shape
    bpw = B // NW
    nch = bpw // CH
    grid2 = grid.reshape(W // 2, 2 * ND)
    mesh = plsc.VectorSubcoreMesh(core_axis_name="c", subcore_axis_name="s")
    body = functools.partial(_spline_body, W, ND, bpw, nch)
    f = pl.kernel(
        body,
        mesh=mesh,
        out_type=jax.ShapeDtypeStruct((B * ND // CH, CH), jnp.float32),
        scratch_types=[
            pltpu.VMEM((bpw,), jnp.float32),
            pltpu.VMEM((bpw,), jnp.int32),
            pltpu.VMEM((bpw,), jnp.int32),
            pltpu.VMEM((bpw,), jnp.int32),
            pltpu.VMEM((bpw,), jnp.int32),
            pltpu.VMEM((bpw,), jnp.float32),
            pltpu.VMEM((bpw,), jnp.float32),
            pltpu.VMEM((bpw,), jnp.float32),
            pltpu.VMEM((bpw,), jnp.float32),
            pltpu.VMEM((3, CH, 2 * ND), jnp.float32),
            pltpu.VMEM((ND, CH + 1), jnp.float32),
            pltpu.SemaphoreType.DMA,
        ],
        compiler_params=pltpu.CompilerParams(needs_layout_passes=False),
    )
    out_flat = f(u, grid2)
    return (out_flat.reshape(ND // 8, B // CH, 8, CH)
            .transpose(1, 3, 0, 2).reshape(B, ND))


# speedup vs baseline: 1.1320x; 1.1320x over previous
import functools

import jax
import jax.numpy as jnp
from jax import lax
from jax.experimental import pallas as pl
from jax.experimental.pallas import tpu as pltpu
from jax.experimental.pallas import tpu_sc as plsc

L = 16
NW = 32
CH = 128


def _spline_body(W, ND, bpw, nch, u_hbm, grid_hbm, out_hbm,
                 u_v, idx_v, w_v, rows_v, out_t, sem):
    nc = 2
    wid = lax.axis_index("s") * nc + lax.axis_index("c")
    base = wid * bpw

    pltpu.sync_copy(u_hbm.at[pl.ds(base, bpw)], u_v)

    def wcomp(g, carry):
        sl = pl.ds(g * L, L)
        uu = jnp.clip(u_v[sl], 0.0, 1.0)
        x = uu * jnp.float32(W - 1)
        idx = jnp.minimum(x.astype(jnp.int32), W - 2)
        t = x - idx.astype(jnp.float32)
        t2 = t * t
        t3 = t2 * t
        sixth = jnp.float32(1.0 / 6.0)
        w0 = sixth * (-t3 + 3.0 * t2 - 3.0 * t + 1.0)
        w1 = sixth * (3.0 * t3 - 6.0 * t2 + 4.0)
        w2 = 0.5 * (-t3 + t2 + t) + sixth
        w3 = sixth * t3
        is_lo = idx == 0
        is_hi = idx == W - 2
        v0 = jnp.where(is_lo, 2.0 * w0, w0)
        v1 = jnp.where(is_hi, w1 - w3, w1)
        v2 = w2 + jnp.where(is_lo, -w0, jnp.where(is_hi, w3, 0.0))
        v3 = w3
        idx_v[0, sl] = jnp.maximum(idx - 1, 0)
        idx_v[1, sl] = idx
        idx_v[2, sl] = idx + 1
        idx_v[3, sl] = jnp.minimum(idx + 2, W - 1)
        w_v[0, sl] = v0
        w_v[1, sl] = v1
        w_v[2, sl] = v2
        w_v[3, sl] = v3
        return carry

    lax.fori_loop(0, bpw // L, wcomp, 0, unroll=2)

    lane = lax.iota(jnp.int32, L)
    cvecs = [jj * L + lane for jj in range(ND // L)]

    for c in range(nch):
        cb = c * CH
        cps = [
            pltpu.async_copy(
                grid_hbm.at[idx_v.at[k, pl.ds(cb, CH)]], rows_v.at[k], sem)
            for k in range(4)
        ]
        for cp in cps:
            cp.wait()

        def scomp(g2, carry):
            wsl = pl.ds(cb + g2 * L, L)
            a0 = w_v[0, wsl]
            a1 = w_v[1, wsl]
            a2 = w_v[2, wsl]
            a3 = w_v[3, wsl]
            for j in range(L):
                b = g2 * L + j
                s0, s1, s2, s3 = a0[j], a1[j], a2[j], a3[j]
                bvec = lane * 0 + b
                for jj in range(ND // L):
                    sl = pl.ds(jj * L, L)
                    acc = (rows_v[0, b, sl] * s0 + rows_v[1, b, sl] * s1
                           + rows_v[2, b, sl] * s2 + rows_v[3, b, sl] * s3)
                    plsc.store_scatter(out_t, [cvecs[jj], bvec], acc)
            return carry

        lax.fori_loop(0, CH // L, scomp, 0)
        rb = wid * nch + c
        for cb8 in range(ND // 8):
            pltpu.sync_copy(
                out_t.at[pl.ds(cb8 * 8, 8), pl.ds(0, CH)],
                out_hbm.at[pl.ds((cb8 * (NW * nch) + rb) * 8, 8), :])


def kernel(u, grid):
    B = u.shape[0]
    W, ND = grid.shape
    bpw = B // NW
    nch = bpw // CH
    mesh = plsc.VectorSubcoreMesh(core_axis_name="c", subcore_axis_name="s")
    body = functools.partial(_spline_body, W, ND, bpw, nch)
    f = pl.kernel(
        body,
        mesh=mesh,
        out_type=jax.ShapeDtypeStruct((B * ND // CH, CH), jnp.float32),
        scratch_types=[
            pltpu.VMEM((bpw,), jnp.float32),
            pltpu.VMEM((4, bpw), jnp.int32),
            pltpu.VMEM((4, bpw), jnp.float32),
            pltpu.VMEM((4, CH, ND), jnp.float32),
            pltpu.VMEM((ND, CH + 1), jnp.float32),
            pltpu.SemaphoreType.DMA,
        ],
        compiler_params=pltpu.CompilerParams(
            use_tc_tiling_on_sc=False, needs_layout_passes=False),
    )
    out2 = f(u, grid)
    return (out2.reshape(ND // 8, B // CH, 8, CH)
            .transpose(1, 3, 0, 2).reshape(B, ND))


# speedup vs baseline: 1.2092x; 1.0681x over previous
import functools

import jax
import jax.numpy as jnp
from jax import lax
from jax.experimental import pallas as pl
from jax.experimental.pallas import tpu as pltpu
from jax.experimental.pallas import tpu_sc as plsc

L = 16
NW = 32
CH = 128


def _spline_body(W, ND, bpw, nch, u_hbm, grid_hbm, out_hbm,
                 u_v, idx_v, w_v, rows_v, out_v, sem0, sem1):
    nc = 2
    wid = lax.axis_index("s") * nc + lax.axis_index("c")
    base = wid * bpw

    pltpu.sync_copy(u_hbm.at[pl.ds(base, bpw)], u_v)

    def wcomp(g, carry):
        sl = pl.ds(g * L, L)
        uu = jnp.clip(u_v[sl], 0.0, 1.0)
        x = uu * jnp.float32(W - 1)
        idx = jnp.minimum(x.astype(jnp.int32), W - 2)
        t = x - idx.astype(jnp.float32)
        t2 = t * t
        t3 = t2 * t
        sixth = jnp.float32(1.0 / 6.0)
        w0 = sixth * (-t3 + 3.0 * t2 - 3.0 * t + 1.0)
        w1 = sixth * (3.0 * t3 - 6.0 * t2 + 4.0)
        w2 = 0.5 * (-t3 + t2 + t) + sixth
        w3 = sixth * t3
        is_lo = idx == 0
        is_hi = idx == W - 2
        v0 = jnp.where(is_lo, 2.0 * w0, w0)
        v1 = jnp.where(is_hi, w1 - w3, w1)
        v2 = w2 + jnp.where(is_lo, -w0, jnp.where(is_hi, w3, 0.0))
        v3 = w3
        idx_v[0, sl] = jnp.maximum(idx - 1, 0)
        idx_v[1, sl] = idx
        idx_v[2, sl] = idx + 1
        idx_v[3, sl] = jnp.minimum(idx + 2, W - 1)
        w_v[0, sl] = v0
        w_v[1, sl] = v1
        w_v[2, sl] = v2
        w_v[3, sl] = v3
        return carry

    lax.fori_loop(0, bpw // L, wcomp, 0, unroll=2)

    sems = (sem0, sem1)

    def fire(c):
        return [
            pltpu.async_copy(
                grid_hbm.at[idx_v.at[k, pl.ds(c * CH, CH)]],
                rows_v.at[c % 2, k], sems[c % 2])
            for k in range(4)
        ]

    inflight = fire(0)
    for c in range(nch):
        for cp in inflight:
            cp.wait()
        if c + 1 < nch:
            inflight = fire(c + 1)

        def scomp(g2, carry, c=c):
            wsl = pl.ds(c * CH + g2 * L, L)
            a0 = w_v[0, wsl]
            a1 = w_v[1, wsl]
            a2 = w_v[2, wsl]
            a3 = w_v[3, wsl]
            for j in range(L):
                b = g2 * L + j
                s0, s1, s2, s3 = a0[j], a1[j], a2[j], a3[j]
                for jj in range(ND // L):
                    sl = pl.ds(jj * L, L)
                    acc = (rows_v[c % 2, 0, b, sl] * s0
                           + rows_v[c % 2, 1, b, sl] * s1
                           + rows_v[c % 2, 2, b, sl] * s2
                           + rows_v[c % 2, 3, b, sl] * s3)
                    out_v[b, sl] = acc
            return carry

        lax.fori_loop(0, CH // L, scomp, 0)
        pltpu.sync_copy(out_v, out_hbm.at[pl.ds(base + c * CH, CH)])


def kernel(u, grid):
    B = u.shape[0]
    W, ND = grid.shape
    bpw = B // NW
    nch = bpw // CH
    mesh = plsc.VectorSubcoreMesh(core_axis_name="c", subcore_axis_name="s")
    body = functools.partial(_spline_body, W, ND, bpw, nch)
    f = pl.kernel(
        body,
        mesh=mesh,
        out_type=jax.ShapeDtypeStruct((B, ND), jnp.float32),
        scratch_types=[
            pltpu.VMEM((bpw,), jnp.float32),
            pltpu.VMEM((4, bpw), jnp.int32),
            pltpu.VMEM((4, bpw), jnp.float32),
            pltpu.VMEM((2, 4, CH, ND), jnp.float32),
            pltpu.VMEM((CH, ND), jnp.float32),
            pltpu.SemaphoreType.DMA,
            pltpu.SemaphoreType.DMA,
        ],
        compiler_params=pltpu.CompilerParams(use_tc_tiling_on_sc=False),
    )
    return f(u, grid)
